# Initial kernel scaffold; baseline (speedup 1.0000x reference)
import jax
import jax.numpy as jnp
from jax.experimental import pallas as pl


def kernel(features, indices, dim, dim_size):
    raise NotImplementedError("write your pallas kernel here")


# speedup vs baseline: 3.4536x; 3.4536x over previous
import functools

import jax
import jax.numpy as jnp
from jax import lax
from jax.experimental import pallas as pl
from jax.experimental.pallas import tpu as pltpu
from jax.experimental.pallas import tpu_sc as plsc

N_EDGES = 320000
D = 128
N_NODES = 10000
CHUNK = 80

NC, NS = 2, 16
NW = NC * NS
E_PER_W = N_EDGES // NW
N_CHUNKS = E_PER_W // CHUNK
N_PAD = 10240
ROWS_PER_TILE = N_PAD // NS
N_SUB = ROWS_PER_TILE // CHUNK

_mesh = plsc.VectorSubcoreMesh(core_axis_name="c", subcore_axis_name="s")


@functools.partial(
    pl.kernel,
    out_type=(
        jax.ShapeDtypeStruct((NC * N_PAD, D), jnp.float32),
        jax.ShapeDtypeStruct((NC * N_PAD, D), jnp.float32),
    ),
    mesh=_mesh,
    scratch_types=[
        pltpu.VMEM((CHUNK,), jnp.int32),
        pltpu.VMEM((CHUNK, D), jnp.float32),
        pltpu.VMEM((CHUNK, D), jnp.float32),
        pltpu.VMEM_SHARED((N_PAD, D), jnp.float32),
    ],
)
def _scatter_stage(feat_hbm, idx_hbm, zsum_hbm, ones_hbm,
                   sums_out, cnts_out,
                   idx_v, rows_v, ones_v, acc_sh):
    c = lax.axis_index("c")
    s = lax.axis_index("s")
    wid = s * NC + c
    r0 = pl.multiple_of(s * ROWS_PER_TILE, 8)
    o0 = pl.multiple_of(c * N_PAD + r0, 8)
    e0 = wid * E_PER_W

    def zero_acc():
        for k in range(N_SUB):
            pltpu.sync_copy(rows_v, acc_sh.at[pl.ds(r0 + k * CHUNK, CHUNK)])

    def write_acc(out_ref):
        for k in range(N_SUB):
            pltpu.sync_copy(acc_sh.at[pl.ds(r0 + k * CHUNK, CHUNK)], rows_v)
            pltpu.sync_copy(rows_v, out_ref.at[pl.ds(o0 + k * CHUNK, CHUNK)])

    pltpu.sync_copy(zsum_hbm, rows_v)
    pltpu.sync_copy(ones_hbm, ones_v)
    zero_acc()
    plsc.subcore_barrier()

    def body_sum(i, carry):
        base = pl.multiple_of(e0 + i * CHUNK, CHUNK)
        pltpu.sync_copy(idx_hbm.at[pl.ds(base, CHUNK)], idx_v)
        pltpu.sync_copy(feat_hbm.at[pl.ds(base, CHUNK)], rows_v)
        pltpu.sync_copy(rows_v, acc_sh.at[idx_v], add=True)
        return carry

    lax.fori_loop(0, N_CHUNKS, body_sum, 0)
    plsc.subcore_barrier()
    write_acc(sums_out)
    plsc.subcore_barrier()

    pltpu.sync_copy(zsum_hbm, rows_v)
    zero_acc()
    plsc.subcore_barrier()

    def body_cnt(i, carry):
        base = pl.multiple_of(e0 + i * CHUNK, CHUNK)
        pltpu.sync_copy(idx_hbm.at[pl.ds(base, CHUNK)], idx_v)
        pltpu.sync_copy(ones_v, acc_sh.at[idx_v], add=True)
        return carry

    lax.fori_loop(0, N_CHUNKS, body_cnt, 0)
    plsc.subcore_barrier()
    write_acc(cnts_out)


_BLK = 1000


def _combine_body(s_ref, c_ref, o_ref):
    total = s_ref[0] + s_ref[1]
    cnt = c_ref[0, :, 0:1] + c_ref[1, :, 0:1]
    mean = total / jnp.maximum(cnt, 1.0)
    o_ref[:, :D] = total
    o_ref[:, D:] = mean


_combine = pl.pallas_call(
    _combine_body,
    grid=(N_NODES // _BLK,),
    in_specs=[
        pl.BlockSpec((NC, _BLK, D), lambda i: (0, i, 0)),
        pl.BlockSpec((NC, _BLK, D), lambda i: (0, i, 0)),
    ],
    out_specs=pl.BlockSpec((_BLK, 2 * D), lambda i: (i, 0)),
    out_shape=jax.ShapeDtypeStruct((N_NODES, 2 * D), jnp.float32),
)


def kernel(features, indices, dim, dim_size):
    del dim, dim_size
    zsum = jnp.zeros((CHUNK, D), jnp.float32)
    ones = jnp.ones((CHUNK, D), jnp.float32)
    sums_p, cnts_p = _scatter_stage(features, indices, zsum, ones)
    return _combine(sums_p.reshape(NC, N_PAD, D), cnts_p.reshape(NC, N_PAD, D))


# speedup vs baseline: 5.6170x; 1.6264x over previous
import functools

import jax
import jax.numpy as jnp
from jax import lax
from jax.experimental import pallas as pl
from jax.experimental.pallas import tpu as pltpu
from jax.experimental.pallas import tpu_sc as plsc

N_EDGES = 320000
D = 128
N_NODES = 10000

NC, NS = 2, 16
NW = NC * NS
CH = 128
E_PAD = 327680
E_PER_W = E_PAD // NW
GROUP = 1024
N_GROUPS = E_PER_W // GROUP
IDX_ROWS = E_PAD // CH
IDX_ROWS_PER_W = E_PER_W // CH
FCLAMP = N_EDGES - CH
N_PAD = 10240
ROWS_PER_TILE = N_PAD // NS
N_SUB = ROWS_PER_TILE // CH

_mesh = plsc.VectorSubcoreMesh(core_axis_name="c", subcore_axis_name="s")


@functools.partial(
    pl.kernel,
    out_type=(
        jax.ShapeDtypeStruct((NC * N_PAD, D), jnp.float32),
        jax.ShapeDtypeStruct((NC * N_PAD, D), jnp.float32),
    ),
    mesh=_mesh,
    scratch_types=[
        pltpu.VMEM((IDX_ROWS_PER_W, CH), jnp.int32),
        pltpu.VMEM((CH, D), jnp.float32),
        pltpu.VMEM((CH, D), jnp.float32),
        pltpu.VMEM_SHARED((N_PAD, D), jnp.float32),
        pltpu.SemaphoreType.DMA,
        pltpu.SemaphoreType.DMA,
        pltpu.SemaphoreType.DMA,
        pltpu.SemaphoreType.DMA,
    ],
)
def _scatter_stage(feat_hbm, idx_hbm, zsum_hbm, ones_hbm,
                   sums_out, cnts_out,
                   idx_v, buf0, buf1, acc_sh,
                   sem_d0, sem_d1, sem_s0, sem_s1):
    c = lax.axis_index("c")
    s = lax.axis_index("s")
    wid = s * NC + c
    r0 = pl.multiple_of(s * ROWS_PER_TILE, 8)
    o0 = pl.multiple_of(c * N_PAD + r0, 8)
    iw = pl.multiple_of(wid * IDX_ROWS_PER_W, 8)
    e0 = wid * E_PER_W

    def zero_acc():
        pltpu.sync_copy(zsum_hbm, buf0)
        for k in range(N_SUB):
            pltpu.sync_copy(buf0, acc_sh.at[pl.ds(r0 + k * CH, CH)])

    def write_acc(out_ref):
        for k in range(N_SUB):
            pltpu.sync_copy(acc_sh.at[pl.ds(r0 + k * CH, CH)], buf0)
            pltpu.sync_copy(buf0, out_ref.at[pl.ds(o0 + k * CH, CH)])

    def fb(j):
        return pl.multiple_of(lax.min(e0 + j * CH, FCLAMP), 8)

    def wait_dma(buf, sem):
        pltpu.make_async_copy(feat_hbm.at[pl.ds(0, CH)], buf, sem).wait()

    pltpu.sync_copy(idx_hbm.at[pl.ds(iw, IDX_ROWS_PER_W)], idx_v)
    zero_acc()
    plsc.subcore_barrier()

    pltpu.async_copy(feat_hbm.at[pl.ds(fb(0), CH)], buf0, sem_d0)
    pltpu.async_copy(feat_hbm.at[pl.ds(fb(1), CH)], buf1, sem_d1)

    def body_sum(t, carry):
        j0 = t * 2
        wait_dma(buf0, sem_d0)
        h0 = pltpu.async_copy(buf0, acc_sh.at[idx_v.at[j0]], sem_s0, add=True)
        wait_dma(buf1, sem_d1)
        h1 = pltpu.async_copy(buf1, acc_sh.at[idx_v.at[j0 + 1]], sem_s1, add=True)
        h0.wait()
        pltpu.async_copy(feat_hbm.at[pl.ds(fb(j0 + 2), CH)], buf0, sem_d0)
        h1.wait()
        pltpu.async_copy(feat_hbm.at[pl.ds(fb(j0 + 3), CH)], buf1, sem_d1)
        return carry

    lax.fori_loop(0, IDX_ROWS_PER_W // 2, body_sum, 0)
    wait_dma(buf0, sem_d0)
    wait_dma(buf1, sem_d1)
    plsc.subcore_barrier()
    write_acc(sums_out)
    plsc.subcore_barrier()

    zero_acc()
    pltpu.sync_copy(ones_hbm, buf1)
    plsc.subcore_barrier()

    def fire_cnt(j):
        return pltpu.async_copy(buf1, acc_sh.at[idx_v.at[j]], sem_s0,
                                add=True)

    def wait_cnt():
        pltpu.make_async_copy(buf1, acc_sh.at[idx_v.at[0]], sem_s0).wait()

    for j in range(4):
        fire_cnt(j)

    def body_cnt(t, carry):
        wait_cnt()
        fire_cnt(t + 4)
        return carry

    lax.fori_loop(0, IDX_ROWS_PER_W - 4, body_cnt, 0)
    for _ in range(4):
        wait_cnt()
    plsc.subcore_barrier()
    write_acc(cnts_out)


_BLK = 1000


def _combine_body(s_ref, c_ref, o_ref):
    total = s_ref[0] + s_ref[1]
    cnt = c_ref[0, :, 0:1] + c_ref[1, :, 0:1]
    mean = total / jnp.maximum(cnt, 1.0)
    o_ref[:, :D] = total
    o_ref[:, D:] = mean


_combine = pl.pallas_call(
    _combine_body,
    grid=(N_NODES // _BLK,),
    in_specs=[
        pl.BlockSpec((NC, _BLK, D), lambda i: (0, i, 0)),
        pl.BlockSpec((NC, _BLK, D), lambda i: (0, i, 0)),
    ],
    out_specs=pl.BlockSpec((_BLK, 2 * D), lambda i: (i, 0)),
    out_shape=jax.ShapeDtypeStruct((N_NODES, 2 * D), jnp.float32),
)


def kernel(features, indices, dim, dim_size):
    del dim, dim_size
    idx_pad = jnp.concatenate(
        [indices, jnp.full((E_PAD - N_EDGES,), N_PAD - 1, jnp.int32)]
    ).reshape(IDX_ROWS, CH)
    zsum = jnp.zeros((CH, D), jnp.float32)
    ones = jnp.ones((CH, D), jnp.float32)
    sums_p, cnts_p = _scatter_stage(features, idx_pad, zsum, ones)
    return _combine(sums_p.reshape(NC, N_PAD, D), cnts_p.reshape(NC, N_PAD, D))


# speedup vs baseline: 5.6848x; 1.0121x over previous
import functools

import jax
import jax.numpy as jnp
from jax import lax
from jax.experimental import pallas as pl
from jax.experimental.pallas import tpu as pltpu
from jax.experimental.pallas import tpu_sc as plsc

N_EDGES = 320000
D = 128
N_NODES = 10000

NC, NS = 2, 16
NW = NC * NS
CH = 128
E_PAD = 327680
E_PER_W = E_PAD // NW
GROUP = 1024
N_GROUPS = E_PER_W // GROUP
IDX_ROWS = E_PAD // CH
IDX_ROWS_PER_W = E_PER_W // CH
FCLAMP = N_EDGES - CH
N_PAD = 10240
ROWS_PER_TILE = N_PAD // NS
N_SUB = ROWS_PER_TILE // CH

_mesh = plsc.VectorSubcoreMesh(core_axis_name="c", subcore_axis_name="s")


@functools.partial(
    pl.kernel,
    out_type=(
        jax.ShapeDtypeStruct((NC * N_PAD, D), jnp.float32),
        jax.ShapeDtypeStruct((NC * N_PAD, D), jnp.float32),
    ),
    mesh=_mesh,
    scratch_types=[
        pltpu.VMEM((IDX_ROWS_PER_W, CH), jnp.int32),
        pltpu.VMEM((CH, D), jnp.float32),
        pltpu.VMEM((CH, D), jnp.float32),
        pltpu.VMEM_SHARED((N_PAD, D), jnp.float32),
        pltpu.SemaphoreType.DMA,
        pltpu.SemaphoreType.DMA,
        pltpu.SemaphoreType.DMA,
        pltpu.SemaphoreType.DMA,
    ],
)
def _scatter_stage(feat_hbm, idx_hbm, zsum_hbm, ones_hbm,
                   sums_out, cnts_out,
                   idx_v, buf0, buf1, acc_sh,
                   sem_d0, sem_d1, sem_s0, sem_s1):
    c = lax.axis_index("c")
    s = lax.axis_index("s")
    wid = s * NC + c
    r0 = pl.multiple_of(s * ROWS_PER_TILE, 8)
    o0 = pl.multiple_of(c * N_PAD + r0, 8)
    iw = pl.multiple_of(wid * IDX_ROWS_PER_W, 8)
    e0 = wid * E_PER_W

    def zero_acc():
        pltpu.sync_copy(zsum_hbm, buf0)
        for k in range(N_SUB):
            pltpu.sync_copy(buf0, acc_sh.at[pl.ds(r0 + k * CH, CH)])

    def write_acc(out_ref):
        bufs = (buf0, buf1)
        sems = (sem_d0, sem_d1)
        h = [None, None]
        for k in range(N_SUB):
            b = k % 2
            if h[b] is not None:
                h[b].wait()
            pltpu.sync_copy(acc_sh.at[pl.ds(r0 + k * CH, CH)], bufs[b])
            h[b] = pltpu.async_copy(bufs[b], out_ref.at[pl.ds(o0 + k * CH, CH)],
                                    sems[b])
        for hh in h:
            if hh is not None:
                hh.wait()

    def fb(j):
        return pl.multiple_of(lax.min(e0 + j * CH, FCLAMP), 8)

    def wait_dma(buf, sem):
        pltpu.make_async_copy(feat_hbm.at[pl.ds(0, CH)], buf, sem).wait()

    pltpu.sync_copy(idx_hbm.at[pl.ds(iw, IDX_ROWS_PER_W)], idx_v)
    zero_acc()
    plsc.subcore_barrier()

    pltpu.async_copy(feat_hbm.at[pl.ds(fb(0), CH)], buf0, sem_d0)
    pltpu.async_copy(feat_hbm.at[pl.ds(fb(1), CH)], buf1, sem_d1)

    def body_sum(t, carry):
        j0 = t * 2
        wait_dma(buf0, sem_d0)
        h0 = pltpu.async_copy(buf0, acc_sh.at[idx_v.at[j0]], sem_s0, add=True)
        wait_dma(buf1, sem_d1)
        h1 = pltpu.async_copy(buf1, acc_sh.at[idx_v.at[j0 + 1]], sem_s1, add=True)
        h0.wait()
        pltpu.async_copy(feat_hbm.at[pl.ds(fb(j0 + 2), CH)], buf0, sem_d0)
        h1.wait()
        pltpu.async_copy(feat_hbm.at[pl.ds(fb(j0 + 3), CH)], buf1, sem_d1)
        return carry

    lax.fori_loop(0, IDX_ROWS_PER_W // 2, body_sum, 0)
    wait_dma(buf0, sem_d0)
    wait_dma(buf1, sem_d1)
    plsc.subcore_barrier()
    write_acc(sums_out)
    plsc.subcore_barrier()

    zero_acc()
    pltpu.sync_copy(ones_hbm, buf1)
    plsc.subcore_barrier()

    def fire_cnt(j):
        return pltpu.async_copy(buf1, acc_sh.at[idx_v.at[j]], sem_s0,
                                add=True)

    def wait_cnt():
        pltpu.make_async_copy(buf1, acc_sh.at[idx_v.at[0]], sem_s0).wait()

    for j in range(8):
        fire_cnt(j)

    def body_cnt(t, carry):
        wait_cnt()
        fire_cnt(t + 8)
        return carry

    lax.fori_loop(0, IDX_ROWS_PER_W - 8, body_cnt, 0)
    for _ in range(8):
        wait_cnt()
    plsc.subcore_barrier()
    write_acc(cnts_out)


_BLK = 1000


def _combine_body(s_ref, c_ref, o_ref):
    total = s_ref[0] + s_ref[1]
    cnt = c_ref[0, :, 0:1] + c_ref[1, :, 0:1]
    mean = total / jnp.maximum(cnt, 1.0)
    o_ref[:, :D] = total
    o_ref[:, D:] = mean


_combine = pl.pallas_call(
    _combine_body,
    grid=(N_NODES // _BLK,),
    in_specs=[
        pl.BlockSpec((NC, _BLK, D), lambda i: (0, i, 0)),
        pl.BlockSpec((NC, _BLK, D), lambda i: (0, i, 0)),
    ],
    out_specs=pl.BlockSpec((_BLK, 2 * D), lambda i: (i, 0)),
    out_shape=jax.ShapeDtypeStruct((N_NODES, 2 * D), jnp.float32),
)


def kernel(features, indices, dim, dim_size):
    del dim, dim_size
    idx_pad = jnp.concatenate(
        [indices, jnp.full((E_PAD - N_EDGES,), N_PAD - 1, jnp.int32)]
    ).reshape(IDX_ROWS, CH)
    zsum = jnp.zeros((CH, D), jnp.float32)
    ones = jnp.ones((CH, D), jnp.float32)
    sums_p, cnts_p = _scatter_stage(features, idx_pad, zsum, ones)
    return _combine(sums_p.reshape(NC, N_PAD, D), cnts_p.reshape(NC, N_PAD, D))
